# Initial kernel scaffold; baseline (speedup 1.0000x reference)
import functools

import jax
import jax.numpy as jnp
from jax import lax
from jax.experimental import pallas as pl
from jax.experimental.pallas import tpu as pltpu
from jax.experimental.pallas import tpu_sc as plsc

N = 10000
F = 128
H = 32
A = 64
E = 320000

NC = 2
NS = 16
L = 16

CH = 128
NCHUNK = 80
EPT = NCHUNK * CH
EPAD = NC * NS * EPT
NPAD = 10240
RPS = NPAD // NS

_mesh = plsc.VectorSubcoreMesh(
    core_axis_name="c", subcore_axis_name="s", num_cores=NC, num_subcores=NS
)


@functools.partial(
    pl.kernel,
    out_type=jax.ShapeDtypeStruct((NC, NPAD), jnp.float32),
    mesh=_mesh,
    scratch_types=[
        pltpu.VMEM((NCHUNK, CH), jnp.int32),
        pltpu.VMEM((CH,), jnp.float32),
        pltpu.VMEM((RPS,), jnp.float32),
        pltpu.VMEM_SHARED((NPAD,), jnp.float32),
        pltpu.SemaphoreType.DMA,
    ],
)
def _sc_deg(dst_hbm, zeros_hbm, out_hbm, didx_v, ones_v, stage_v, accum_sh, sem):
    c = lax.axis_index("c")
    s = lax.axis_index("s")
    pltpu.sync_copy(dst_hbm.at[c, s], didx_v)
    for k in range(CH // L):
        ones_v[pl.ds(k * L, L)] = jnp.full((L,), 1.0, jnp.float32)
    pltpu.sync_copy(zeros_hbm.at[pl.ds(s * RPS, RPS)], accum_sh.at[pl.ds(s * RPS, RPS)])
    plsc.subcore_barrier()

    def chunk(j, carry):
        pltpu.sync_copy(ones_v, accum_sh.at[didx_v.at[j]], add=True)
        return carry

    lax.fori_loop(0, NCHUNK, chunk, 0)
    plsc.subcore_barrier()
    pltpu.sync_copy(accum_sh.at[pl.ds(s * RPS, RPS)], stage_v)
    pltpu.sync_copy(stage_v, out_hbm.at[c, pl.ds(s * RPS, RPS)])


@functools.partial(
    pl.kernel,
    out_type=jax.ShapeDtypeStruct((NC, NPAD, H), jnp.float32),
    mesh=_mesh,
    scratch_types=[
        pltpu.VMEM((NCHUNK, CH), jnp.int32),
        pltpu.VMEM((NCHUNK, CH), jnp.int32),
        pltpu.VMEM((CH, H), jnp.float32),
        pltpu.VMEM((RPS, H), jnp.float32),
        pltpu.VMEM_SHARED((NPAD, H), jnp.float32),
        pltpu.SemaphoreType.DMA,
    ],
)
def _sc_prop(src_hbm, dst_hbm, g_hbm, zeros_hbm, out_hbm,
             sidx_v, didx_v, rows_v, stage_v, accum_sh, sem):
    c = lax.axis_index("c")
    s = lax.axis_index("s")
    pltpu.sync_copy(src_hbm.at[c, s], sidx_v)
    pltpu.sync_copy(dst_hbm.at[c, s], didx_v)
    pltpu.sync_copy(zeros_hbm.at[pl.ds(s * RPS, RPS)], accum_sh.at[pl.ds(s * RPS, RPS)])
    plsc.subcore_barrier()

    def chunk(j, carry):
        pltpu.async_copy(g_hbm.at[sidx_v.at[j]], rows_v, sem).wait()
        pltpu.sync_copy(rows_v, accum_sh.at[didx_v.at[j]], add=True)
        return carry

    lax.fori_loop(0, NCHUNK, chunk, 0)
    plsc.subcore_barrier()
    pltpu.sync_copy(accum_sh.at[pl.ds(s * RPS, RPS)], stage_v)
    pltpu.sync_copy(stage_v, out_hbm.at[c, pl.ds(s * RPS, RPS)])


R = 1000
GRID = N // R


def _pre_body(x_ref, w1_ref, dpart_ref, dinv_ref, z_ref, g_ref):
    d = dpart_ref[0] + dpart_ref[1]
    dinv = jax.lax.rsqrt(1.0 + d)
    z = jnp.dot(x_ref[...], w1_ref[...], preferred_element_type=jnp.float32)
    dinv_ref[...] = dinv
    z_ref[...] = z
    g_ref[...] = z * dinv


_pre_call = pl.pallas_call(
    _pre_body,
    grid=(GRID,),
    in_specs=[
        pl.BlockSpec((R, F), lambda i: (i, 0)),
        pl.BlockSpec((F, H), lambda i: (0, 0)),
        pl.BlockSpec((NC, R, 1), lambda i: (0, i, 0)),
    ],
    out_specs=[
        pl.BlockSpec((R, 1), lambda i: (i, 0)),
        pl.BlockSpec((R, H), lambda i: (i, 0)),
        pl.BlockSpec((R, H), lambda i: (i, 0)),
    ],
    out_shape=[
        jax.ShapeDtypeStruct((N, 1), jnp.float32),
        jax.ShapeDtypeStruct((N, H), jnp.float32),
        jax.ShapeDtypeStruct((N, H), jnp.float32),
    ],
)


def _mid_body(z_ref, p_ref, dinv_ref, b1_ref, w2_ref, z2_ref, g2_ref):
    dinv = dinv_ref[...]
    scat = p_ref[0] + p_ref[1]
    h = jnp.maximum(dinv * scat + dinv * dinv * z_ref[...] + b1_ref[...], 0.0)
    z2 = jnp.dot(h, w2_ref[...], preferred_element_type=jnp.float32)
    z2_ref[...] = z2
    g2_ref[...] = z2 * dinv


_mid_call = pl.pallas_call(
    _mid_body,
    grid=(GRID,),
    in_specs=[
        pl.BlockSpec((R, H), lambda i: (i, 0)),
        pl.BlockSpec((NC, R, H), lambda i: (0, i, 0)),
        pl.BlockSpec((R, 1), lambda i: (i, 0)),
        pl.BlockSpec((1, H), lambda i: (0, 0)),
        pl.BlockSpec((H, H), lambda i: (0, 0)),
    ],
    out_specs=[
        pl.BlockSpec((R, H), lambda i: (i, 0)),
        pl.BlockSpec((R, H), lambda i: (i, 0)),
    ],
    out_shape=[
        jax.ShapeDtypeStruct((N, H), jnp.float32),
        jax.ShapeDtypeStruct((N, H), jnp.float32),
    ],
)


def _post_body(z_ref, p_ref, dinv_ref, b2_ref, p1_ref, pb1_ref, p2_ref, pb2_ref, out_ref):
    dinv = dinv_ref[...]
    scat = p_ref[0] + p_ref[1]
    h = jnp.maximum(dinv * scat + dinv * dinv * z_ref[...] + b2_ref[...], 0.0)
    hidden = jnp.maximum(
        jnp.dot(h, p1_ref[...], preferred_element_type=jnp.float32) + pb1_ref[...], 0.0
    )
    out_ref[...] = (
        jnp.dot(hidden, p2_ref[...], preferred_element_type=jnp.float32) + pb2_ref[...]
    )


_post_call = pl.pallas_call(
    _post_body,
    grid=(GRID,),
    in_specs=[
        pl.BlockSpec((R, H), lambda i: (i, 0)),
        pl.BlockSpec((NC, R, H), lambda i: (0, i, 0)),
        pl.BlockSpec((R, 1), lambda i: (i, 0)),
        pl.BlockSpec((1, H), lambda i: (0, 0)),
        pl.BlockSpec((H, 512), lambda i: (0, 0)),
        pl.BlockSpec((1, 512), lambda i: (0, 0)),
        pl.BlockSpec((512, A), lambda i: (0, 0)),
        pl.BlockSpec((1, A), lambda i: (0, 0)),
    ],
    out_specs=pl.BlockSpec((R, A), lambda i: (i, 0)),
    out_shape=jax.ShapeDtypeStruct((N, A), jnp.float32),
)


def kernel(features, edge_index, W1, b1, W2, b2, P1, pb1, P2, pb2):
    src = edge_index[0]
    dst = edge_index[1]
    npad = EPAD - E
    src_p = jnp.concatenate([src, jnp.zeros((npad,), jnp.int32)]).reshape(NC, NS, NCHUNK, CH)
    dst_p = jnp.concatenate([dst, jnp.full((npad,), N, jnp.int32)]).reshape(NC, NS, NCHUNK, CH)
    zeros1 = jnp.zeros((NPAD,), jnp.float32)
    zerosH = jnp.zeros((NPAD, H), jnp.float32)

    dpart = _sc_deg(dst_p, zeros1)
    dpart3 = dpart[:, :N].reshape(NC, N, 1)
    dinv, z1, g1 = _pre_call(features, W1, dpart3)
    pp1 = _sc_prop(src_p, dst_p, g1, zerosH)
    z2, g2 = _mid_call(z1, pp1[:, :N], dinv, b1.reshape(1, H), W2)
    pp2 = _sc_prop(src_p, dst_p, g2, zerosH)
    policy = _post_call(
        z2, pp2[:, :N], dinv, b2.reshape(1, H),
        P1, pb1.reshape(1, 512), P2, pb2.reshape(1, A),
    )
    return policy


# speedup vs baseline: 19.4772x; 19.4772x over previous
import functools

import jax
import jax.numpy as jnp
from jax import lax
from jax.experimental import pallas as pl
from jax.experimental.pallas import tpu as pltpu
from jax.experimental.pallas import tpu_sc as plsc

N = 10000
F = 128
H = 32
A = 64
E = 320000

NC = 2
NS = 16
L = 16

CH = 128
NCHUNK = 80
EPT = NCHUNK * CH
EPAD = NC * NS * EPT
NPAD = 10240
RPS = NPAD // NS

_mesh = plsc.VectorSubcoreMesh(
    core_axis_name="c", subcore_axis_name="s", num_cores=NC, num_subcores=NS
)


@functools.partial(
    pl.kernel,
    out_type=jax.ShapeDtypeStruct((NC, NPAD), jnp.float32),
    mesh=_mesh,
    scratch_types=[
        pltpu.VMEM((NCHUNK, CH), jnp.int32),
        pltpu.VMEM((CH,), jnp.float32),
        pltpu.VMEM((RPS,), jnp.float32),
        pltpu.VMEM_SHARED((NPAD,), jnp.float32),
        pltpu.SemaphoreType.DMA,
    ],
    compiler_params=pltpu.CompilerParams(use_tc_tiling_on_sc=False),
)
def _sc_deg(dst_hbm, zeros_hbm, out_hbm, didx_v, ones_v, stage_v, accum_sh, sem):
    c = lax.axis_index("c")
    s = lax.axis_index("s")
    pltpu.sync_copy(dst_hbm.at[c, s], didx_v)
    for k in range(CH // L):
        ones_v[pl.ds(k * L, L)] = jnp.full((L,), 1.0, jnp.float32)
    pltpu.sync_copy(zeros_hbm.at[pl.ds(s * RPS, RPS)], accum_sh.at[pl.ds(s * RPS, RPS)])
    plsc.subcore_barrier()

    def chunk(j, carry):
        pltpu.sync_copy(ones_v, accum_sh.at[didx_v.at[j]], add=True)
        return carry

    lax.fori_loop(0, NCHUNK, chunk, 0)
    plsc.subcore_barrier()
    pltpu.sync_copy(accum_sh.at[pl.ds(s * RPS, RPS)], stage_v)
    pltpu.sync_copy(stage_v, out_hbm.at[c, pl.ds(s * RPS, RPS)])


@functools.partial(
    pl.kernel,
    out_type=jax.ShapeDtypeStruct((NC, NPAD, H), jnp.float32),
    mesh=_mesh,
    scratch_types=[
        pltpu.VMEM((NCHUNK, CH), jnp.int32),
        pltpu.VMEM((NCHUNK, CH), jnp.int32),
        pltpu.VMEM((CH, H), jnp.float32),
        pltpu.VMEM((RPS, H), jnp.float32),
        pltpu.VMEM_SHARED((NPAD, H), jnp.float32),
        pltpu.SemaphoreType.DMA,
    ],
    compiler_params=pltpu.CompilerParams(use_tc_tiling_on_sc=False),
)
def _sc_prop(src_hbm, dst_hbm, g_hbm, zeros_hbm, out_hbm,
             sidx_v, didx_v, rows_v, stage_v, accum_sh, sem):
    c = lax.axis_index("c")
    s = lax.axis_index("s")
    pltpu.sync_copy(src_hbm.at[c, s], sidx_v)
    pltpu.sync_copy(dst_hbm.at[c, s], didx_v)
    pltpu.sync_copy(zeros_hbm.at[pl.ds(s * RPS, RPS)], accum_sh.at[pl.ds(s * RPS, RPS)])
    plsc.subcore_barrier()

    def chunk(j, carry):
        pltpu.async_copy(g_hbm.at[sidx_v.at[j]], rows_v, sem).wait()
        pltpu.sync_copy(rows_v, accum_sh.at[didx_v.at[j]], add=True)
        return carry

    lax.fori_loop(0, NCHUNK, chunk, 0)
    plsc.subcore_barrier()
    pltpu.sync_copy(accum_sh.at[pl.ds(s * RPS, RPS)], stage_v)
    pltpu.sync_copy(stage_v, out_hbm.at[c, pl.ds(s * RPS, RPS)])


R = 1000
GRID = N // R


def _pre_body(x_ref, w1_ref, dpart_ref, dinv_ref, z_ref, g_ref):
    d = dpart_ref[0] + dpart_ref[1]
    dinv = jax.lax.rsqrt(1.0 + d)
    z = jnp.dot(x_ref[...], w1_ref[...], preferred_element_type=jnp.float32)
    dinv_ref[...] = dinv
    z_ref[...] = z
    g_ref[...] = z * dinv


_pre_call = pl.pallas_call(
    _pre_body,
    grid=(GRID,),
    in_specs=[
        pl.BlockSpec((R, F), lambda i: (i, 0)),
        pl.BlockSpec((F, H), lambda i: (0, 0)),
        pl.BlockSpec((NC, R, 1), lambda i: (0, i, 0)),
    ],
    out_specs=[
        pl.BlockSpec((R, 1), lambda i: (i, 0)),
        pl.BlockSpec((R, H), lambda i: (i, 0)),
        pl.BlockSpec((R, H), lambda i: (i, 0)),
    ],
    out_shape=[
        jax.ShapeDtypeStruct((N, 1), jnp.float32),
        jax.ShapeDtypeStruct((N, H), jnp.float32),
        jax.ShapeDtypeStruct((N, H), jnp.float32),
    ],
)


def _mid_body(z_ref, p_ref, dinv_ref, b1_ref, w2_ref, z2_ref, g2_ref):
    dinv = dinv_ref[...]
    scat = p_ref[0] + p_ref[1]
    h = jnp.maximum(dinv * scat + dinv * dinv * z_ref[...] + b1_ref[...], 0.0)
    z2 = jnp.dot(h, w2_ref[...], preferred_element_type=jnp.float32)
    z2_ref[...] = z2
    g2_ref[...] = z2 * dinv


_mid_call = pl.pallas_call(
    _mid_body,
    grid=(GRID,),
    in_specs=[
        pl.BlockSpec((R, H), lambda i: (i, 0)),
        pl.BlockSpec((NC, R, H), lambda i: (0, i, 0)),
        pl.BlockSpec((R, 1), lambda i: (i, 0)),
        pl.BlockSpec((1, H), lambda i: (0, 0)),
        pl.BlockSpec((H, H), lambda i: (0, 0)),
    ],
    out_specs=[
        pl.BlockSpec((R, H), lambda i: (i, 0)),
        pl.BlockSpec((R, H), lambda i: (i, 0)),
    ],
    out_shape=[
        jax.ShapeDtypeStruct((N, H), jnp.float32),
        jax.ShapeDtypeStruct((N, H), jnp.float32),
    ],
)


def _post_body(z_ref, p_ref, dinv_ref, b2_ref, p1_ref, pb1_ref, p2_ref, pb2_ref, out_ref):
    dinv = dinv_ref[...]
    scat = p_ref[0] + p_ref[1]
    h = jnp.maximum(dinv * scat + dinv * dinv * z_ref[...] + b2_ref[...], 0.0)
    hidden = jnp.maximum(
        jnp.dot(h, p1_ref[...], preferred_element_type=jnp.float32) + pb1_ref[...], 0.0
    )
    out_ref[...] = (
        jnp.dot(hidden, p2_ref[...], preferred_element_type=jnp.float32) + pb2_ref[...]
    )


_post_call = pl.pallas_call(
    _post_body,
    grid=(GRID,),
    in_specs=[
        pl.BlockSpec((R, H), lambda i: (i, 0)),
        pl.BlockSpec((NC, R, H), lambda i: (0, i, 0)),
        pl.BlockSpec((R, 1), lambda i: (i, 0)),
        pl.BlockSpec((1, H), lambda i: (0, 0)),
        pl.BlockSpec((H, 512), lambda i: (0, 0)),
        pl.BlockSpec((1, 512), lambda i: (0, 0)),
        pl.BlockSpec((512, A), lambda i: (0, 0)),
        pl.BlockSpec((1, A), lambda i: (0, 0)),
    ],
    out_specs=pl.BlockSpec((R, A), lambda i: (i, 0)),
    out_shape=jax.ShapeDtypeStruct((N, A), jnp.float32),
)


def kernel(features, edge_index, W1, b1, W2, b2, P1, pb1, P2, pb2):
    src = edge_index[0]
    dst = edge_index[1]
    npad = EPAD - E
    src_p = jnp.concatenate([src, jnp.zeros((npad,), jnp.int32)]).reshape(NC, NS, NCHUNK, CH)
    dst_p = jnp.concatenate([dst, jnp.full((npad,), N, jnp.int32)]).reshape(NC, NS, NCHUNK, CH)
    zeros1 = jnp.zeros((NPAD,), jnp.float32)
    zerosH = jnp.zeros((NPAD, H), jnp.float32)

    dpart = _sc_deg(dst_p, zeros1)
    dpart3 = dpart[:, :N].reshape(NC, N, 1)
    dinv, z1, g1 = _pre_call(features, W1, dpart3)
    pp1 = _sc_prop(src_p, dst_p, g1, zerosH)
    z2, g2 = _mid_call(z1, pp1[:, :N], dinv, b1.reshape(1, H), W2)
    pp2 = _sc_prop(src_p, dst_p, g2, zerosH)
    policy = _post_call(
        z2, pp2[:, :N], dinv, b2.reshape(1, H),
        P1, pb1.reshape(1, 512), P2, pb2.reshape(1, A),
    )
    return policy


# speedup vs baseline: 24.0127x; 1.2329x over previous
import functools

import jax
import jax.numpy as jnp
from jax import lax
from jax.experimental import pallas as pl
from jax.experimental.pallas import tpu as pltpu
from jax.experimental.pallas import tpu_sc as plsc

N = 10000
F = 128
H = 32
A = 64
E = 320000

NC = 2
NS = 16
L = 16

CH = 128
NCHUNK = 80
EPT = NCHUNK * CH
EPAD = NC * NS * EPT
NPAD = 10240
RPS = NPAD // NS

_mesh = plsc.VectorSubcoreMesh(
    core_axis_name="c", subcore_axis_name="s", num_cores=NC, num_subcores=NS
)


@functools.partial(
    pl.kernel,
    out_type=jax.ShapeDtypeStruct((NC, NPAD), jnp.float32),
    mesh=_mesh,
    scratch_types=[
        pltpu.VMEM((NCHUNK, CH), jnp.int32),
        pltpu.VMEM((CH,), jnp.float32),
        pltpu.VMEM((RPS,), jnp.float32),
        pltpu.VMEM_SHARED((NPAD,), jnp.float32),
        pltpu.SemaphoreType.DMA,
    ],
    compiler_params=pltpu.CompilerParams(use_tc_tiling_on_sc=False),
)
def _sc_deg(dst_hbm, zeros_hbm, out_hbm, didx_v, ones_v, stage_v, accum_sh, sem):
    c = lax.axis_index("c")
    s = lax.axis_index("s")
    pltpu.sync_copy(dst_hbm.at[c, s], didx_v)
    for k in range(CH // L):
        ones_v[pl.ds(k * L, L)] = jnp.full((L,), 1.0, jnp.float32)
    pltpu.sync_copy(zeros_hbm.at[pl.ds(s * RPS, RPS)], accum_sh.at[pl.ds(s * RPS, RPS)])
    plsc.subcore_barrier()

    def chunk(j, carry):
        pltpu.sync_copy(ones_v, accum_sh.at[didx_v.at[j]], add=True)
        return carry

    lax.fori_loop(0, NCHUNK, chunk, 0)
    plsc.subcore_barrier()
    pltpu.sync_copy(accum_sh.at[pl.ds(s * RPS, RPS)], stage_v)
    pltpu.sync_copy(stage_v, out_hbm.at[c, pl.ds(s * RPS, RPS)])


@functools.partial(
    pl.kernel,
    out_type=jax.ShapeDtypeStruct((NC, NPAD, H), jnp.float32),
    mesh=_mesh,
    scratch_types=[
        pltpu.VMEM((NCHUNK, CH), jnp.int32),
        pltpu.VMEM((NCHUNK, CH), jnp.int32),
        pltpu.VMEM((CH, H), jnp.float32),
        pltpu.VMEM((CH, H), jnp.float32),
        pltpu.VMEM((RPS, H), jnp.float32),
        pltpu.VMEM_SHARED((NPAD, H), jnp.float32),
        pltpu.SemaphoreType.DMA,
        pltpu.SemaphoreType.DMA,
    ],
    compiler_params=pltpu.CompilerParams(use_tc_tiling_on_sc=False),
)
def _sc_prop(src_hbm, dst_hbm, g_hbm, zeros_hbm, out_hbm,
             sidx_v, didx_v, rows0_v, rows1_v, stage_v, accum_sh, sem0, sem1):
    c = lax.axis_index("c")
    s = lax.axis_index("s")
    pltpu.sync_copy(src_hbm.at[c, s], sidx_v)
    pltpu.sync_copy(dst_hbm.at[c, s], didx_v)
    pltpu.sync_copy(zeros_hbm.at[pl.ds(s * RPS, RPS)], accum_sh.at[pl.ds(s * RPS, RPS)])
    plsc.subcore_barrier()

    pltpu.async_copy(g_hbm.at[sidx_v.at[0]], rows0_v, sem0)

    def chunk(i, carry):
        j = 2 * i
        pltpu.async_copy(g_hbm.at[sidx_v.at[j + 1]], rows1_v, sem1)
        pltpu.make_async_copy(g_hbm.at[sidx_v.at[0]], rows0_v, sem0).wait()
        pltpu.sync_copy(rows0_v, accum_sh.at[didx_v.at[j]], add=True)

        @pl.when(j + 2 < NCHUNK)
        def _():
            pltpu.async_copy(g_hbm.at[sidx_v.at[j + 2]], rows0_v, sem0)

        pltpu.make_async_copy(g_hbm.at[sidx_v.at[0]], rows1_v, sem1).wait()
        pltpu.sync_copy(rows1_v, accum_sh.at[didx_v.at[j + 1]], add=True)
        return carry

    lax.fori_loop(0, NCHUNK // 2, chunk, 0)
    plsc.subcore_barrier()
    pltpu.sync_copy(accum_sh.at[pl.ds(s * RPS, RPS)], stage_v)
    pltpu.sync_copy(stage_v, out_hbm.at[c, pl.ds(s * RPS, RPS)])


R = 1000
GRID = N // R


def _pre_body(x_ref, w1_ref, dpart_ref, dinv_ref, z_ref, g_ref):
    d = dpart_ref[0] + dpart_ref[1]
    dinv = jax.lax.rsqrt(1.0 + d)
    z = jnp.dot(x_ref[...], w1_ref[...], preferred_element_type=jnp.float32)
    dinv_ref[...] = dinv
    z_ref[...] = z
    g_ref[...] = z * dinv


_pre_call = pl.pallas_call(
    _pre_body,
    grid=(GRID,),
    in_specs=[
        pl.BlockSpec((R, F), lambda i: (i, 0)),
        pl.BlockSpec((F, H), lambda i: (0, 0)),
        pl.BlockSpec((NC, R, 1), lambda i: (0, i, 0)),
    ],
    out_specs=[
        pl.BlockSpec((R, 1), lambda i: (i, 0)),
        pl.BlockSpec((R, H), lambda i: (i, 0)),
        pl.BlockSpec((R, H), lambda i: (i, 0)),
    ],
    out_shape=[
        jax.ShapeDtypeStruct((N, 1), jnp.float32),
        jax.ShapeDtypeStruct((N, H), jnp.float32),
        jax.ShapeDtypeStruct((N, H), jnp.float32),
    ],
)


def _mid_body(z_ref, p_ref, dinv_ref, b1_ref, w2_ref, z2_ref, g2_ref):
    dinv = dinv_ref[...]
    scat = p_ref[0] + p_ref[1]
    h = jnp.maximum(dinv * scat + dinv * dinv * z_ref[...] + b1_ref[...], 0.0)
    z2 = jnp.dot(h, w2_ref[...], preferred_element_type=jnp.float32)
    z2_ref[...] = z2
    g2_ref[...] = z2 * dinv


_mid_call = pl.pallas_call(
    _mid_body,
    grid=(GRID,),
    in_specs=[
        pl.BlockSpec((R, H), lambda i: (i, 0)),
        pl.BlockSpec((NC, R, H), lambda i: (0, i, 0)),
        pl.BlockSpec((R, 1), lambda i: (i, 0)),
        pl.BlockSpec((1, H), lambda i: (0, 0)),
        pl.BlockSpec((H, H), lambda i: (0, 0)),
    ],
    out_specs=[
        pl.BlockSpec((R, H), lambda i: (i, 0)),
        pl.BlockSpec((R, H), lambda i: (i, 0)),
    ],
    out_shape=[
        jax.ShapeDtypeStruct((N, H), jnp.float32),
        jax.ShapeDtypeStruct((N, H), jnp.float32),
    ],
)


def _post_body(z_ref, p_ref, dinv_ref, b2_ref, p1_ref, pb1_ref, p2_ref, pb2_ref, out_ref):
    dinv = dinv_ref[...]
    scat = p_ref[0] + p_ref[1]
    h = jnp.maximum(dinv * scat + dinv * dinv * z_ref[...] + b2_ref[...], 0.0)
    hidden = jnp.maximum(
        jnp.dot(h, p1_ref[...], preferred_element_type=jnp.float32) + pb1_ref[...], 0.0
    )
    out_ref[...] = (
        jnp.dot(hidden, p2_ref[...], preferred_element_type=jnp.float32) + pb2_ref[...]
    )


_post_call = pl.pallas_call(
    _post_body,
    grid=(GRID,),
    in_specs=[
        pl.BlockSpec((R, H), lambda i: (i, 0)),
        pl.BlockSpec((NC, R, H), lambda i: (0, i, 0)),
        pl.BlockSpec((R, 1), lambda i: (i, 0)),
        pl.BlockSpec((1, H), lambda i: (0, 0)),
        pl.BlockSpec((H, 512), lambda i: (0, 0)),
        pl.BlockSpec((1, 512), lambda i: (0, 0)),
        pl.BlockSpec((512, A), lambda i: (0, 0)),
        pl.BlockSpec((1, A), lambda i: (0, 0)),
    ],
    out_specs=pl.BlockSpec((R, A), lambda i: (i, 0)),
    out_shape=jax.ShapeDtypeStruct((N, A), jnp.float32),
)


def kernel(features, edge_index, W1, b1, W2, b2, P1, pb1, P2, pb2):
    src = edge_index[0]
    dst = edge_index[1]
    npad = EPAD - E
    src_p = jnp.concatenate([src, jnp.zeros((npad,), jnp.int32)]).reshape(NC, NS, NCHUNK, CH)
    dst_p = jnp.concatenate([dst, jnp.full((npad,), N, jnp.int32)]).reshape(NC, NS, NCHUNK, CH)
    zeros1 = jnp.zeros((NPAD,), jnp.float32)
    zerosH = jnp.zeros((NPAD, H), jnp.float32)

    dpart = _sc_deg(dst_p, zeros1)
    dpart3 = dpart[:, :N].reshape(NC, N, 1)
    dinv, z1, g1 = _pre_call(features, W1, dpart3)
    pp1 = _sc_prop(src_p, dst_p, g1, zerosH)
    z2, g2 = _mid_call(z1, pp1[:, :N], dinv, b1.reshape(1, H), W2)
    pp2 = _sc_prop(src_p, dst_p, g2, zerosH)
    policy = _post_call(
        z2, pp2[:, :N], dinv, b2.reshape(1, H),
        P1, pb1.reshape(1, 512), P2, pb2.reshape(1, A),
    )
    return policy


# speedup vs baseline: 42.0021x; 1.7492x over previous
import functools

import jax
import jax.numpy as jnp
from jax import lax
from jax.experimental import pallas as pl
from jax.experimental.pallas import tpu as pltpu
from jax.experimental.pallas import tpu_sc as plsc

N = 10000
F = 128
H = 32
A = 64
E = 320000

NC = 2
NS = 16
L = 16

CH = 128
NCHUNK = 80
EPT = NCHUNK * CH
EPAD = NC * NS * EPT
NPAD = 10240
RPS = NPAD // NS

_mesh = plsc.VectorSubcoreMesh(
    core_axis_name="c", subcore_axis_name="s", num_cores=NC, num_subcores=NS
)


@functools.partial(
    pl.kernel,
    out_type=jax.ShapeDtypeStruct((NC, NPAD), jnp.float32),
    mesh=_mesh,
    scratch_types=[
        pltpu.VMEM((NCHUNK, CH), jnp.int32),
        pltpu.VMEM((CH,), jnp.float32),
        pltpu.VMEM((RPS,), jnp.float32),
        pltpu.VMEM_SHARED((NPAD,), jnp.float32),
        pltpu.SemaphoreType.DMA,
    ],
    compiler_params=pltpu.CompilerParams(use_tc_tiling_on_sc=False),
)
def _sc_deg(dst_hbm, zeros_hbm, out_hbm, didx_v, ones_v, stage_v, accum_sh, sem):
    c = lax.axis_index("c")
    s = lax.axis_index("s")
    pltpu.sync_copy(dst_hbm.at[c, s], didx_v)
    for k in range(CH // L):
        ones_v[pl.ds(k * L, L)] = jnp.full((L,), 1.0, jnp.float32)
    pltpu.sync_copy(zeros_hbm.at[pl.ds(s * RPS, RPS)], accum_sh.at[pl.ds(s * RPS, RPS)])
    plsc.subcore_barrier()

    def chunk(j, carry):
        pltpu.sync_copy(ones_v, accum_sh.at[didx_v.at[j]], add=True)
        return carry

    lax.fori_loop(0, NCHUNK, chunk, 0)
    plsc.subcore_barrier()
    pltpu.sync_copy(accum_sh.at[pl.ds(s * RPS, RPS)], stage_v)
    pltpu.sync_copy(stage_v, out_hbm.at[c, pl.ds(s * RPS, RPS)])


@functools.partial(
    pl.kernel,
    out_type=jax.ShapeDtypeStruct((NC, NPAD, H), jnp.float32),
    mesh=_mesh,
    scratch_types=[
        pltpu.VMEM((NCHUNK, CH), jnp.int32),
        pltpu.VMEM((NCHUNK, CH), jnp.int32),
        pltpu.VMEM((CH, H), jnp.float32),
        pltpu.VMEM((CH, H), jnp.float32),
        pltpu.VMEM((RPS, H), jnp.float32),
        pltpu.VMEM_SHARED((NPAD, H), jnp.float32),
        pltpu.VMEM_SHARED((N, H), jnp.float32),
        pltpu.SemaphoreType.DMA,
        pltpu.SemaphoreType.DMA,
    ],
    compiler_params=pltpu.CompilerParams(use_tc_tiling_on_sc=False),
)
def _sc_prop(src_hbm, dst_hbm, g_hbm, zeros_hbm, out_hbm,
             sidx_v, didx_v, rows0_v, rows1_v, stage_v, accum_sh, g_sh, sem0, sem1):
    c = lax.axis_index("c")
    s = lax.axis_index("s")
    pltpu.sync_copy(src_hbm.at[c, s], sidx_v)
    pltpu.sync_copy(dst_hbm.at[c, s], didx_v)
    pltpu.sync_copy(zeros_hbm.at[pl.ds(s * RPS, RPS)], accum_sh.at[pl.ds(s * RPS, RPS)])
    pltpu.sync_copy(g_hbm.at[pl.ds(s * (N // NS), N // NS)],
                    g_sh.at[pl.ds(s * (N // NS), N // NS)])
    plsc.subcore_barrier()

    pltpu.async_copy(g_sh.at[sidx_v.at[0]], rows0_v, sem0)

    def chunk(i, carry):
        j = 2 * i
        pltpu.async_copy(g_sh.at[sidx_v.at[j + 1]], rows1_v, sem1)
        pltpu.make_async_copy(g_sh.at[sidx_v.at[0]], rows0_v, sem0).wait()
        pltpu.sync_copy(rows0_v, accum_sh.at[didx_v.at[j]], add=True)

        @pl.when(j + 2 < NCHUNK)
        def _():
            pltpu.async_copy(g_sh.at[sidx_v.at[j + 2]], rows0_v, sem0)

        pltpu.make_async_copy(g_sh.at[sidx_v.at[0]], rows1_v, sem1).wait()
        pltpu.sync_copy(rows1_v, accum_sh.at[didx_v.at[j + 1]], add=True)
        return carry

    lax.fori_loop(0, NCHUNK // 2, chunk, 0)
    plsc.subcore_barrier()
    pltpu.sync_copy(accum_sh.at[pl.ds(s * RPS, RPS)], stage_v)
    pltpu.sync_copy(stage_v, out_hbm.at[c, pl.ds(s * RPS, RPS)])


R = 1000
GRID = N // R


def _pre_body(x_ref, w1_ref, dpart_ref, dinv_ref, z_ref, g_ref):
    d = dpart_ref[0] + dpart_ref[1]
    dinv = jax.lax.rsqrt(1.0 + d)
    z = jnp.dot(x_ref[...], w1_ref[...], preferred_element_type=jnp.float32)
    dinv_ref[...] = dinv
    z_ref[...] = z
    g_ref[...] = z * dinv


_pre_call = pl.pallas_call(
    _pre_body,
    grid=(GRID,),
    in_specs=[
        pl.BlockSpec((R, F), lambda i: (i, 0)),
        pl.BlockSpec((F, H), lambda i: (0, 0)),
        pl.BlockSpec((NC, R, 1), lambda i: (0, i, 0)),
    ],
    out_specs=[
        pl.BlockSpec((R, 1), lambda i: (i, 0)),
        pl.BlockSpec((R, H), lambda i: (i, 0)),
        pl.BlockSpec((R, H), lambda i: (i, 0)),
    ],
    out_shape=[
        jax.ShapeDtypeStruct((N, 1), jnp.float32),
        jax.ShapeDtypeStruct((N, H), jnp.float32),
        jax.ShapeDtypeStruct((N, H), jnp.float32),
    ],
)


def _mid_body(z_ref, p_ref, dinv_ref, b1_ref, w2_ref, z2_ref, g2_ref):
    dinv = dinv_ref[...]
    scat = p_ref[0] + p_ref[1]
    h = jnp.maximum(dinv * scat + dinv * dinv * z_ref[...] + b1_ref[...], 0.0)
    z2 = jnp.dot(h, w2_ref[...], preferred_element_type=jnp.float32)
    z2_ref[...] = z2
    g2_ref[...] = z2 * dinv


_mid_call = pl.pallas_call(
    _mid_body,
    grid=(GRID,),
    in_specs=[
        pl.BlockSpec((R, H), lambda i: (i, 0)),
        pl.BlockSpec((NC, R, H), lambda i: (0, i, 0)),
        pl.BlockSpec((R, 1), lambda i: (i, 0)),
        pl.BlockSpec((1, H), lambda i: (0, 0)),
        pl.BlockSpec((H, H), lambda i: (0, 0)),
    ],
    out_specs=[
        pl.BlockSpec((R, H), lambda i: (i, 0)),
        pl.BlockSpec((R, H), lambda i: (i, 0)),
    ],
    out_shape=[
        jax.ShapeDtypeStruct((N, H), jnp.float32),
        jax.ShapeDtypeStruct((N, H), jnp.float32),
    ],
)


def _post_body(z_ref, p_ref, dinv_ref, b2_ref, p1_ref, pb1_ref, p2_ref, pb2_ref, out_ref):
    dinv = dinv_ref[...]
    scat = p_ref[0] + p_ref[1]
    h = jnp.maximum(dinv * scat + dinv * dinv * z_ref[...] + b2_ref[...], 0.0)
    hidden = jnp.maximum(
        jnp.dot(h, p1_ref[...], preferred_element_type=jnp.float32) + pb1_ref[...], 0.0
    )
    out_ref[...] = (
        jnp.dot(hidden, p2_ref[...], preferred_element_type=jnp.float32) + pb2_ref[...]
    )


_post_call = pl.pallas_call(
    _post_body,
    grid=(GRID,),
    in_specs=[
        pl.BlockSpec((R, H), lambda i: (i, 0)),
        pl.BlockSpec((NC, R, H), lambda i: (0, i, 0)),
        pl.BlockSpec((R, 1), lambda i: (i, 0)),
        pl.BlockSpec((1, H), lambda i: (0, 0)),
        pl.BlockSpec((H, 512), lambda i: (0, 0)),
        pl.BlockSpec((1, 512), lambda i: (0, 0)),
        pl.BlockSpec((512, A), lambda i: (0, 0)),
        pl.BlockSpec((1, A), lambda i: (0, 0)),
    ],
    out_specs=pl.BlockSpec((R, A), lambda i: (i, 0)),
    out_shape=jax.ShapeDtypeStruct((N, A), jnp.float32),
)


def kernel(features, edge_index, W1, b1, W2, b2, P1, pb1, P2, pb2):
    src = edge_index[0]
    dst = edge_index[1]
    npad = EPAD - E
    src_p = jnp.concatenate([src, jnp.zeros((npad,), jnp.int32)]).reshape(NC, NS, NCHUNK, CH)
    dst_p = jnp.concatenate([dst, jnp.full((npad,), N, jnp.int32)]).reshape(NC, NS, NCHUNK, CH)
    zeros1 = jnp.zeros((NPAD,), jnp.float32)
    zerosH = jnp.zeros((NPAD, H), jnp.float32)

    dpart = _sc_deg(dst_p, zeros1)
    dpart3 = dpart[:, :N].reshape(NC, N, 1)
    dinv, z1, g1 = _pre_call(features, W1, dpart3)
    pp1 = _sc_prop(src_p, dst_p, g1, zerosH)
    z2, g2 = _mid_call(z1, pp1[:, :N], dinv, b1.reshape(1, H), W2)
    pp2 = _sc_prop(src_p, dst_p, g2, zerosH)
    policy = _post_call(
        z2, pp2[:, :N], dinv, b2.reshape(1, H),
        P1, pb1.reshape(1, 512), P2, pb2.reshape(1, A),
    )
    return policy


# speedup vs baseline: 43.4926x; 1.0355x over previous
import functools

import jax
import jax.numpy as jnp
from jax import lax
from jax.experimental import pallas as pl
from jax.experimental.pallas import tpu as pltpu
from jax.experimental.pallas import tpu_sc as plsc

N = 10000
F = 128
H = 32
A = 64
E = 320000

NC = 2
NS = 16
L = 16
W = NC * NS

CH = 100
NCHUNK = 100
EPT = NCHUNK * CH
NB = 4

NPS = N // NS
NPAD = 10240
RPS = NPAD // NS

_mesh = plsc.VectorSubcoreMesh(
    core_axis_name="c", subcore_axis_name="s", num_cores=NC, num_subcores=NS
)


@functools.partial(
    pl.kernel,
    out_type=jax.ShapeDtypeStruct((NC, NPAD), jnp.float32),
    mesh=_mesh,
    scratch_types=[
        pltpu.VMEM((NCHUNK, CH), jnp.int32),
        pltpu.VMEM((112,), jnp.float32),
        pltpu.VMEM((RPS,), jnp.float32),
        pltpu.VMEM_SHARED((NPAD,), jnp.float32),
        pltpu.SemaphoreType.DMA,
    ],
    compiler_params=pltpu.CompilerParams(use_tc_tiling_on_sc=False),
)
def _sc_deg(dst_hbm, zeros_hbm, out_hbm, didx_v, ones_v, stage_v, accum_sh, sem):
    c = lax.axis_index("c")
    s = lax.axis_index("s")
    w = c * NS + s
    pltpu.sync_copy(dst_hbm.at[w], didx_v)
    for k in range(7):
        ones_v[pl.ds(k * L, L)] = jnp.full((L,), 1.0, jnp.float32)
    pltpu.sync_copy(zeros_hbm.at[pl.ds(s * RPS, RPS)], accum_sh.at[pl.ds(s * RPS, RPS)])
    plsc.subcore_barrier()

    def fire(j, carry):
        pltpu.async_copy(ones_v.at[pl.ds(0, CH)], accum_sh.at[didx_v.at[j]], sem, add=True)
        return carry

    lax.fori_loop(0, NCHUNK, fire, 0)

    def drain(j, carry):
        pltpu.make_async_copy(ones_v.at[pl.ds(0, CH)], accum_sh.at[didx_v.at[0]], sem).wait()
        return carry

    lax.fori_loop(0, NCHUNK, drain, 0)
    plsc.subcore_barrier()
    pltpu.sync_copy(accum_sh.at[pl.ds(s * RPS, RPS)], stage_v)
    pltpu.sync_copy(stage_v, out_hbm.at[c, pl.ds(s * RPS, RPS)])


@functools.partial(
    pl.kernel,
    out_type=jax.ShapeDtypeStruct((NC, N, H), jnp.float32),
    mesh=_mesh,
    scratch_types=[
        pltpu.VMEM((NCHUNK, CH), jnp.int32),
        pltpu.VMEM((NCHUNK, CH), jnp.int32),
        pltpu.VMEM((CH, H), jnp.float32),
        pltpu.VMEM((CH, H), jnp.float32),
        pltpu.VMEM((CH, H), jnp.float32),
        pltpu.VMEM((CH, H), jnp.float32),
        pltpu.VMEM((NPS, H), jnp.float32),
        pltpu.VMEM_SHARED((N, H), jnp.float32),
        pltpu.VMEM_SHARED((N, H), jnp.float32),
        pltpu.SemaphoreType.DMA,
        pltpu.SemaphoreType.DMA,
        pltpu.SemaphoreType.DMA,
        pltpu.SemaphoreType.DMA,
        pltpu.SemaphoreType.DMA,
        pltpu.SemaphoreType.DMA,
        pltpu.SemaphoreType.DMA,
        pltpu.SemaphoreType.DMA,
    ],
    compiler_params=pltpu.CompilerParams(use_tc_tiling_on_sc=False),
)
def _sc_prop(src_hbm, dst_hbm, g_hbm, zeros_hbm, out_hbm,
             sidx_v, didx_v, r0, r1, r2, r3, stage_v, accum_sh, g_sh,
             g0, g1, g2, g3, s0, s1, s2, s3):
    c = lax.axis_index("c")
    s = lax.axis_index("s")
    w = c * NS + s
    bufs = (r0, r1, r2, r3)
    gsem = (g0, g1, g2, g3)
    ssem = (s0, s1, s2, s3)

    pltpu.sync_copy(src_hbm.at[w], sidx_v)
    pltpu.sync_copy(dst_hbm.at[w], didx_v)
    pltpu.sync_copy(zeros_hbm.at[pl.ds(s * NPS, NPS)], accum_sh.at[pl.ds(s * NPS, NPS)])
    pltpu.sync_copy(g_hbm.at[pl.ds(s * NPS, NPS)], g_sh.at[pl.ds(s * NPS, NPS)])
    plsc.subcore_barrier()

    for b in range(NB):
        pltpu.async_copy(g_sh.at[sidx_v.at[b]], bufs[b], gsem[b])

    def group(i, carry):
        j0 = NB * i
        for b in range(NB):
            pltpu.make_async_copy(g_sh.at[sidx_v.at[0]], bufs[b], gsem[b]).wait()
            pltpu.async_copy(bufs[b], accum_sh.at[didx_v.at[j0 + b]], ssem[b], add=True)
        for b in range(NB):
            pltpu.make_async_copy(bufs[b], accum_sh.at[didx_v.at[0]], ssem[b]).wait()
            pltpu.async_copy(g_sh.at[sidx_v.at[j0 + NB + b]], bufs[b], gsem[b])
        return carry

    lax.fori_loop(0, NCHUNK // NB - 1, group, 0)
    j0 = NCHUNK - NB
    for b in range(NB):
        pltpu.make_async_copy(g_sh.at[sidx_v.at[0]], bufs[b], gsem[b]).wait()
        pltpu.async_copy(bufs[b], accum_sh.at[didx_v.at[j0 + b]], ssem[b], add=True)
    for b in range(NB):
        pltpu.make_async_copy(bufs[b], accum_sh.at[didx_v.at[0]], ssem[b]).wait()

    plsc.subcore_barrier()
    pltpu.sync_copy(accum_sh.at[pl.ds(s * NPS, NPS)], stage_v)
    pltpu.sync_copy(stage_v, out_hbm.at[c, pl.ds(s * NPS, NPS)])


R = 1000
GRID = N // R


def _z1_body(x_ref, w1_ref, z_ref):
    z_ref[...] = jnp.dot(x_ref[...], w1_ref[...], preferred_element_type=jnp.float32)


_z1_call = pl.pallas_call(
    _z1_body,
    grid=(GRID,),
    in_specs=[
        pl.BlockSpec((R, F), lambda i: (i, 0)),
        pl.BlockSpec((F, H), lambda i: (0, 0)),
    ],
    out_specs=pl.BlockSpec((R, H), lambda i: (i, 0)),
    out_shape=jax.ShapeDtypeStruct((N, H), jnp.float32),
)


def _pre_body(z_ref, dpart_ref, dinv_ref, g_ref):
    d = dpart_ref[0] + dpart_ref[1]
    dinv = jax.lax.rsqrt(1.0 + d)
    dinv_ref[...] = dinv
    g_ref[...] = z_ref[...] * dinv


_pre_call = pl.pallas_call(
    _pre_body,
    grid=(GRID,),
    in_specs=[
        pl.BlockSpec((R, H), lambda i: (i, 0)),
        pl.BlockSpec((NC, R, 1), lambda i: (0, i, 0)),
    ],
    out_specs=[
        pl.BlockSpec((R, 1), lambda i: (i, 0)),
        pl.BlockSpec((R, H), lambda i: (i, 0)),
    ],
    out_shape=[
        jax.ShapeDtypeStruct((N, 1), jnp.float32),
        jax.ShapeDtypeStruct((N, H), jnp.float32),
    ],
)


def _mid_body(z_ref, p_ref, dinv_ref, b1_ref, w2_ref, z2_ref, g2_ref):
    dinv = dinv_ref[...]
    scat = p_ref[0] + p_ref[1]
    h = jnp.maximum(dinv * scat + dinv * dinv * z_ref[...] + b1_ref[...], 0.0)
    z2 = jnp.dot(h, w2_ref[...], preferred_element_type=jnp.float32)
    z2_ref[...] = z2
    g2_ref[...] = z2 * dinv


_mid_call = pl.pallas_call(
    _mid_body,
    grid=(GRID,),
    in_specs=[
        pl.BlockSpec((R, H), lambda i: (i, 0)),
        pl.BlockSpec((NC, R, H), lambda i: (0, i, 0)),
        pl.BlockSpec((R, 1), lambda i: (i, 0)),
        pl.BlockSpec((1, H), lambda i: (0, 0)),
        pl.BlockSpec((H, H), lambda i: (0, 0)),
    ],
    out_specs=[
        pl.BlockSpec((R, H), lambda i: (i, 0)),
        pl.BlockSpec((R, H), lambda i: (i, 0)),
    ],
    out_shape=[
        jax.ShapeDtypeStruct((N, H), jnp.float32),
        jax.ShapeDtypeStruct((N, H), jnp.float32),
    ],
)


def _post_body(z_ref, p_ref, dinv_ref, b2_ref, p1_ref, pb1_ref, p2_ref, pb2_ref, out_ref):
    dinv = dinv_ref[...]
    scat = p_ref[0] + p_ref[1]
    h = jnp.maximum(dinv * scat + dinv * dinv * z_ref[...] + b2_ref[...], 0.0)
    hidden = jnp.maximum(
        jnp.dot(h, p1_ref[...], preferred_element_type=jnp.float32) + pb1_ref[...], 0.0
    )
    out_ref[...] = (
        jnp.dot(hidden, p2_ref[...], preferred_element_type=jnp.float32) + pb2_ref[...]
    )


_post_call = pl.pallas_call(
    _post_body,
    grid=(GRID,),
    in_specs=[
        pl.BlockSpec((R, H), lambda i: (i, 0)),
        pl.BlockSpec((NC, R, H), lambda i: (0, i, 0)),
        pl.BlockSpec((R, 1), lambda i: (i, 0)),
        pl.BlockSpec((1, H), lambda i: (0, 0)),
        pl.BlockSpec((H, 512), lambda i: (0, 0)),
        pl.BlockSpec((1, 512), lambda i: (0, 0)),
        pl.BlockSpec((512, A), lambda i: (0, 0)),
        pl.BlockSpec((1, A), lambda i: (0, 0)),
    ],
    out_specs=pl.BlockSpec((R, A), lambda i: (i, 0)),
    out_shape=jax.ShapeDtypeStruct((N, A), jnp.float32),
)


def kernel(features, edge_index, W1, b1, W2, b2, P1, pb1, P2, pb2):
    src3 = edge_index[0].reshape(W, NCHUNK, CH)
    dst3 = edge_index[1].reshape(W, NCHUNK, CH)
    zeros1 = jnp.zeros((NPAD,), jnp.float32)
    zerosH = jnp.zeros((N, H), jnp.float32)

    dpart = _sc_deg(dst3, zeros1)
    z1 = _z1_call(features, W1)
    dinv, g1 = _pre_call(z1, dpart[:, :N].reshape(NC, N, 1))
    pp1 = _sc_prop(src3, dst3, g1, zerosH)
    z2, g2 = _mid_call(z1, pp1, dinv, b1.reshape(1, H), W2)
    pp2 = _sc_prop(src3, dst3, g2, zerosH)
    policy = _post_call(
        z2, pp2, dinv, b2.reshape(1, H),
        P1, pb1.reshape(1, 512), P2, pb2.reshape(1, A),
    )
    return policy


# speedup vs baseline: 44.7491x; 1.0289x over previous
import functools

import jax
import jax.numpy as jnp
from jax import lax
from jax.experimental import pallas as pl
from jax.experimental.pallas import tpu as pltpu
from jax.experimental.pallas import tpu_sc as plsc

N = 10000
F = 128
H = 32
A = 64
E = 320000

NC = 2
NS = 16
L = 16
W = NC * NS

CH = 100
NCHUNK = 100
EPT = NCHUNK * CH
NB = 4

NN = 10240
NPS = NN // NS

_mesh = plsc.VectorSubcoreMesh(
    core_axis_name="c", subcore_axis_name="s", num_cores=NC, num_subcores=NS
)


@functools.partial(
    pl.kernel,
    out_type=jax.ShapeDtypeStruct((NC, NN), jnp.float32),
    mesh=_mesh,
    scratch_types=[
        pltpu.VMEM((NCHUNK, CH), jnp.int32),
        pltpu.VMEM((112,), jnp.float32),
        pltpu.VMEM((NPS,), jnp.float32),
        pltpu.VMEM_SHARED((NN,), jnp.float32),
        pltpu.SemaphoreType.DMA,
    ],
    compiler_params=pltpu.CompilerParams(use_tc_tiling_on_sc=False),
)
def _sc_deg(dst_hbm, zeros_hbm, out_hbm, didx_v, ones_v, stage_v, accum_sh, sem):
    c = lax.axis_index("c")
    s = lax.axis_index("s")
    w = c * NS + s
    pltpu.sync_copy(dst_hbm.at[w], didx_v)
    for k in range(7):
        ones_v[pl.ds(k * L, L)] = jnp.full((L,), 1.0, jnp.float32)
    pltpu.sync_copy(zeros_hbm.at[pl.ds(s * NPS, NPS)], accum_sh.at[pl.ds(s * NPS, NPS)])
    plsc.subcore_barrier()

    def fire(j, carry):
        pltpu.async_copy(ones_v.at[pl.ds(0, CH)], accum_sh.at[didx_v.at[j]], sem, add=True)
        return carry

    lax.fori_loop(0, NCHUNK, fire, 0)

    def drain(j, carry):
        pltpu.make_async_copy(ones_v.at[pl.ds(0, CH)], accum_sh.at[didx_v.at[0]], sem).wait()
        return carry

    lax.fori_loop(0, NCHUNK, drain, 0)
    plsc.subcore_barrier()
    pltpu.sync_copy(accum_sh.at[pl.ds(s * NPS, NPS)], stage_v)
    pltpu.sync_copy(stage_v, out_hbm.at[c, pl.ds(s * NPS, NPS)])


@functools.partial(
    pl.kernel,
    out_type=jax.ShapeDtypeStruct((NC, NN, H), jnp.float32),
    mesh=_mesh,
    scratch_types=[
        pltpu.VMEM((NCHUNK, CH), jnp.int32),
        pltpu.VMEM((NCHUNK, CH), jnp.int32),
        pltpu.VMEM((CH, H), jnp.float32),
        pltpu.VMEM((CH, H), jnp.float32),
        pltpu.VMEM((CH, H), jnp.float32),
        pltpu.VMEM((CH, H), jnp.float32),
        pltpu.VMEM((NPS, H), jnp.float32),
        pltpu.VMEM_SHARED((NN, H), jnp.float32),
        pltpu.VMEM_SHARED((NN, H), jnp.float32),
        pltpu.SemaphoreType.DMA,
        pltpu.SemaphoreType.DMA,
        pltpu.SemaphoreType.DMA,
        pltpu.SemaphoreType.DMA,
        pltpu.SemaphoreType.DMA,
        pltpu.SemaphoreType.DMA,
        pltpu.SemaphoreType.DMA,
        pltpu.SemaphoreType.DMA,
    ],
    compiler_params=pltpu.CompilerParams(use_tc_tiling_on_sc=False),
)
def _sc_prop(src_hbm, dst_hbm, g_hbm, zeros_hbm, out_hbm,
             sidx_v, didx_v, r0, r1, r2, r3, stage_v, accum_sh, g_sh,
             g0, g1, g2, g3, s0, s1, s2, s3):
    c = lax.axis_index("c")
    s = lax.axis_index("s")
    w = c * NS + s
    bufs = (r0, r1, r2, r3)
    gsem = (g0, g1, g2, g3)
    ssem = (s0, s1, s2, s3)

    pltpu.sync_copy(src_hbm.at[w], sidx_v)
    pltpu.sync_copy(dst_hbm.at[w], didx_v)
    pltpu.sync_copy(zeros_hbm.at[pl.ds(s * NPS, NPS)], accum_sh.at[pl.ds(s * NPS, NPS)])
    pltpu.sync_copy(g_hbm.at[pl.ds(s * NPS, NPS)], g_sh.at[pl.ds(s * NPS, NPS)])
    plsc.subcore_barrier()

    for b in range(NB):
        pltpu.async_copy(g_sh.at[sidx_v.at[b]], bufs[b], gsem[b])

    def group(i, carry):
        j0 = NB * i
        for b in range(NB):
            pltpu.make_async_copy(g_sh.at[sidx_v.at[0]], bufs[b], gsem[b]).wait()
            pltpu.async_copy(bufs[b], accum_sh.at[didx_v.at[j0 + b]], ssem[b], add=True)
        for b in range(NB):
            pltpu.make_async_copy(bufs[b], accum_sh.at[didx_v.at[0]], ssem[b]).wait()
            pltpu.async_copy(g_sh.at[sidx_v.at[j0 + NB + b]], bufs[b], gsem[b])
        return carry

    lax.fori_loop(0, NCHUNK // NB - 1, group, 0)
    j0 = NCHUNK - NB
    for b in range(NB):
        pltpu.make_async_copy(g_sh.at[sidx_v.at[0]], bufs[b], gsem[b]).wait()
        pltpu.async_copy(bufs[b], accum_sh.at[didx_v.at[j0 + b]], ssem[b], add=True)
    for b in range(NB):
        pltpu.make_async_copy(bufs[b], accum_sh.at[didx_v.at[0]], ssem[b]).wait()

    plsc.subcore_barrier()
    pltpu.sync_copy(accum_sh.at[pl.ds(s * NPS, NPS)], stage_v)
    pltpu.sync_copy(stage_v, out_hbm.at[c, pl.ds(s * NPS, NPS)])


R = 1024
GRID = NN // R


def _col(dinv_lane):
    return jnp.transpose(dinv_lane.reshape(1, R))


def _z1_body(x_ref, w1_ref, z_ref):
    z_ref[...] = jnp.dot(x_ref[...], w1_ref[...], preferred_element_type=jnp.float32)


_z1_call = pl.pallas_call(
    _z1_body,
    grid=(GRID,),
    in_specs=[
        pl.BlockSpec((R, F), lambda i: (i, 0)),
        pl.BlockSpec((F, H), lambda i: (0, 0)),
    ],
    out_specs=pl.BlockSpec((R, H), lambda i: (i, 0)),
    out_shape=jax.ShapeDtypeStruct((NN, H), jnp.float32),
)


def _pre_body(z_ref, dpart_ref, dinv_ref, g_ref):
    d = dpart_ref[0] + dpart_ref[1]
    dinv = jax.lax.rsqrt(1.0 + d)
    dinv_ref[...] = dinv
    g_ref[...] = z_ref[...] * _col(dinv)


_pre_call = pl.pallas_call(
    _pre_body,
    grid=(GRID,),
    in_specs=[
        pl.BlockSpec((R, H), lambda i: (i, 0)),
        pl.BlockSpec((NC, R), lambda i: (0, i)),
    ],
    out_specs=[
        pl.BlockSpec((R,), lambda i: (i,)),
        pl.BlockSpec((R, H), lambda i: (i, 0)),
    ],
    out_shape=[
        jax.ShapeDtypeStruct((NN,), jnp.float32),
        jax.ShapeDtypeStruct((NN, H), jnp.float32),
    ],
)


def _mid_body(z_ref, p_ref, dinv_ref, b1_ref, w2_ref, z2_ref, g2_ref):
    dcol = _col(dinv_ref[...])
    scat = p_ref[0] + p_ref[1]
    h = jnp.maximum(dcol * scat + dcol * dcol * z_ref[...] + b1_ref[...], 0.0)
    z2 = jnp.dot(h, w2_ref[...], preferred_element_type=jnp.float32)
    z2_ref[...] = z2
    g2_ref[...] = z2 * dcol


_mid_call = pl.pallas_call(
    _mid_body,
    grid=(GRID,),
    in_specs=[
        pl.BlockSpec((R, H), lambda i: (i, 0)),
        pl.BlockSpec((NC, R, H), lambda i: (0, i, 0)),
        pl.BlockSpec((R,), lambda i: (i,)),
        pl.BlockSpec((1, H), lambda i: (0, 0)),
        pl.BlockSpec((H, H), lambda i: (0, 0)),
    ],
    out_specs=[
        pl.BlockSpec((R, H), lambda i: (i, 0)),
        pl.BlockSpec((R, H), lambda i: (i, 0)),
    ],
    out_shape=[
        jax.ShapeDtypeStruct((NN, H), jnp.float32),
        jax.ShapeDtypeStruct((NN, H), jnp.float32),
    ],
)


def _post_body(z_ref, p_ref, dinv_ref, b2_ref, p1_ref, pb1_ref, p2_ref, pb2_ref, out_ref):
    dcol = _col(dinv_ref[...])
    scat = p_ref[0] + p_ref[1]
    h = jnp.maximum(dcol * scat + dcol * dcol * z_ref[...] + b2_ref[...], 0.0)
    hidden = jnp.maximum(
        jnp.dot(h, p1_ref[...], preferred_element_type=jnp.float32) + pb1_ref[...], 0.0
    )
    out_ref[...] = (
        jnp.dot(hidden, p2_ref[...], preferred_element_type=jnp.float32) + pb2_ref[...]
    )


_post_call = pl.pallas_call(
    _post_body,
    grid=(GRID,),
    in_specs=[
        pl.BlockSpec((R, H), lambda i: (i, 0)),
        pl.BlockSpec((NC, R, H), lambda i: (0, i, 0)),
        pl.BlockSpec((R,), lambda i: (i,)),
        pl.BlockSpec((1, H), lambda i: (0, 0)),
        pl.BlockSpec((H, 512), lambda i: (0, 0)),
        pl.BlockSpec((1, 512), lambda i: (0, 0)),
        pl.BlockSpec((512, A), lambda i: (0, 0)),
        pl.BlockSpec((1, A), lambda i: (0, 0)),
    ],
    out_specs=pl.BlockSpec((R, A), lambda i: (i, 0)),
    out_shape=jax.ShapeDtypeStruct((N, A), jnp.float32),
)


def kernel(features, edge_index, W1, b1, W2, b2, P1, pb1, P2, pb2):
    src3 = edge_index[0].reshape(W, NCHUNK, CH)
    dst3 = edge_index[1].reshape(W, NCHUNK, CH)
    zeros1 = jnp.zeros((NN,), jnp.float32)
    zerosH = jnp.zeros((NN, H), jnp.float32)

    dpart = _sc_deg(dst3, zeros1)
    z1 = _z1_call(features, W1)
    dinv, g1 = _pre_call(z1, dpart)
    pp1 = _sc_prop(src3, dst3, g1, zerosH)
    z2, g2 = _mid_call(z1, pp1, dinv, b1.reshape(1, H), W2)
    pp2 = _sc_prop(src3, dst3, g2, zerosH)
    policy = _post_call(
        z2, pp2, dinv, b2.reshape(1, H),
        P1, pb1.reshape(1, 512), P2, pb2.reshape(1, A),
    )
    return policy


# speedup vs baseline: 45.1798x; 1.0096x over previous
import functools

import jax
import jax.numpy as jnp
from jax import lax
from jax.experimental import pallas as pl
from jax.experimental.pallas import tpu as pltpu
from jax.experimental.pallas import tpu_sc as plsc

N = 10000
F = 128
H = 32
A = 64
E = 320000

NC = 2
NS = 16
L = 16
W = NC * NS

CH = 125
NCHUNK = 80
EPT = NCHUNK * CH
NB = 4

NN = 10240
NPS = NN // NS

_mesh = plsc.VectorSubcoreMesh(
    core_axis_name="c", subcore_axis_name="s", num_cores=NC, num_subcores=NS
)


@functools.partial(
    pl.kernel,
    out_type=jax.ShapeDtypeStruct((NC, NN), jnp.float32),
    mesh=_mesh,
    scratch_types=[
        pltpu.VMEM((NCHUNK, CH), jnp.int32),
        pltpu.VMEM((128,), jnp.float32),
        pltpu.VMEM((NPS,), jnp.float32),
        pltpu.VMEM_SHARED((NN,), jnp.float32),
        pltpu.SemaphoreType.DMA,
    ],
    compiler_params=pltpu.CompilerParams(use_tc_tiling_on_sc=False),
)
def _sc_deg(dst_hbm, zeros_hbm, out_hbm, didx_v, ones_v, stage_v, accum_sh, sem):
    c = lax.axis_index("c")
    s = lax.axis_index("s")
    w = c * NS + s
    pltpu.sync_copy(dst_hbm.at[w], didx_v)
    for k in range(8):
        ones_v[pl.ds(k * L, L)] = jnp.full((L,), 1.0, jnp.float32)
    pltpu.sync_copy(zeros_hbm.at[pl.ds(s * NPS, NPS)], accum_sh.at[pl.ds(s * NPS, NPS)])
    plsc.subcore_barrier()

    def fire(j, carry):
        pltpu.async_copy(ones_v.at[pl.ds(0, CH)], accum_sh.at[didx_v.at[j]], sem, add=True)
        return carry

    lax.fori_loop(0, NCHUNK, fire, 0)

    def drain(j, carry):
        pltpu.make_async_copy(ones_v.at[pl.ds(0, CH)], accum_sh.at[didx_v.at[0]], sem).wait()
        return carry

    lax.fori_loop(0, NCHUNK, drain, 0)
    plsc.subcore_barrier()
    pltpu.sync_copy(accum_sh.at[pl.ds(s * NPS, NPS)], stage_v)
    pltpu.sync_copy(stage_v, out_hbm.at[c, pl.ds(s * NPS, NPS)])


@functools.partial(
    pl.kernel,
    out_type=jax.ShapeDtypeStruct((NC, NN, H), jnp.float32),
    mesh=_mesh,
    scratch_types=[
        pltpu.VMEM((NCHUNK, CH), jnp.int32),
        pltpu.VMEM((NCHUNK, CH), jnp.int32),
        pltpu.VMEM((CH, H), jnp.float32),
        pltpu.VMEM((CH, H), jnp.float32),
        pltpu.VMEM((CH, H), jnp.float32),
        pltpu.VMEM((CH, H), jnp.float32),
        pltpu.VMEM((NPS, H), jnp.float32),
        pltpu.VMEM_SHARED((NN, H), jnp.float32),
        pltpu.VMEM_SHARED((NN, H), jnp.float32),
        pltpu.SemaphoreType.DMA,
        pltpu.SemaphoreType.DMA,
        pltpu.SemaphoreType.DMA,
        pltpu.SemaphoreType.DMA,
        pltpu.SemaphoreType.DMA,
        pltpu.SemaphoreType.DMA,
        pltpu.SemaphoreType.DMA,
        pltpu.SemaphoreType.DMA,
    ],
    compiler_params=pltpu.CompilerParams(use_tc_tiling_on_sc=False),
)
def _sc_prop(src_hbm, dst_hbm, g_hbm, zeros_hbm, out_hbm,
             sidx_v, didx_v, r0, r1, r2, r3, stage_v, accum_sh, g_sh,
             g0, g1, g2, g3, s0, s1, s2, s3):
    c = lax.axis_index("c")
    s = lax.axis_index("s")
    w = c * NS + s
    bufs = (r0, r1, r2, r3)
    gsem = (g0, g1, g2, g3)
    ssem = (s0, s1, s2, s3)

    pltpu.sync_copy(src_hbm.at[w], sidx_v)
    pltpu.sync_copy(dst_hbm.at[w], didx_v)
    pltpu.sync_copy(zeros_hbm.at[pl.ds(s * NPS, NPS)], accum_sh.at[pl.ds(s * NPS, NPS)])
    pltpu.sync_copy(g_hbm.at[pl.ds(s * NPS, NPS)], g_sh.at[pl.ds(s * NPS, NPS)])
    plsc.subcore_barrier()

    for b in range(NB):
        pltpu.async_copy(g_sh.at[sidx_v.at[b]], bufs[b], gsem[b])

    def group(i, carry):
        j0 = NB * i
        for b in range(NB):
            pltpu.make_async_copy(g_sh.at[sidx_v.at[0]], bufs[b], gsem[b]).wait()
            pltpu.async_copy(bufs[b], accum_sh.at[didx_v.at[j0 + b]], ssem[b], add=True)
        for b in range(NB):
            pltpu.make_async_copy(bufs[b], accum_sh.at[didx_v.at[0]], ssem[b]).wait()
            pltpu.async_copy(g_sh.at[sidx_v.at[j0 + NB + b]], bufs[b], gsem[b])
        return carry

    lax.fori_loop(0, NCHUNK // NB - 1, group, 0)
    j0 = NCHUNK - NB
    for b in range(NB):
        pltpu.make_async_copy(g_sh.at[sidx_v.at[0]], bufs[b], gsem[b]).wait()
        pltpu.async_copy(bufs[b], accum_sh.at[didx_v.at[j0 + b]], ssem[b], add=True)
    for b in range(NB):
        pltpu.make_async_copy(bufs[b], accum_sh.at[didx_v.at[0]], ssem[b]).wait()

    plsc.subcore_barrier()
    pltpu.sync_copy(accum_sh.at[pl.ds(s * NPS, NPS)], stage_v)
    pltpu.sync_copy(stage_v, out_hbm.at[c, pl.ds(s * NPS, NPS)])


R = 1024
GRID = NN // R


def _col(dinv_lane):
    return jnp.transpose(dinv_lane.reshape(1, R))


def _z1_body(x_ref, w1_ref, z_ref):
    z_ref[...] = jnp.dot(x_ref[...], w1_ref[...], preferred_element_type=jnp.float32)


_z1_call = pl.pallas_call(
    _z1_body,
    grid=(GRID,),
    in_specs=[
        pl.BlockSpec((R, F), lambda i: (i, 0)),
        pl.BlockSpec((F, H), lambda i: (0, 0)),
    ],
    out_specs=pl.BlockSpec((R, H), lambda i: (i, 0)),
    out_shape=jax.ShapeDtypeStruct((NN, H), jnp.float32),
)


def _pre_body(z_ref, dpart_ref, dinv_ref, g_ref):
    d = dpart_ref[0] + dpart_ref[1]
    dinv = jax.lax.rsqrt(1.0 + d)
    dinv_ref[...] = dinv
    g_ref[...] = z_ref[...] * _col(dinv)


_pre_call = pl.pallas_call(
    _pre_body,
    grid=(GRID,),
    in_specs=[
        pl.BlockSpec((R, H), lambda i: (i, 0)),
        pl.BlockSpec((NC, R), lambda i: (0, i)),
    ],
    out_specs=[
        pl.BlockSpec((R,), lambda i: (i,)),
        pl.BlockSpec((R, H), lambda i: (i, 0)),
    ],
    out_shape=[
        jax.ShapeDtypeStruct((NN,), jnp.float32),
        jax.ShapeDtypeStruct((NN, H), jnp.float32),
    ],
)


def _mid_body(z_ref, p_ref, dinv_ref, b1_ref, w2_ref, z2_ref, g2_ref):
    dcol = _col(dinv_ref[...])
    scat = p_ref[0] + p_ref[1]
    h = jnp.maximum(dcol * scat + dcol * dcol * z_ref[...] + b1_ref[...], 0.0)
    z2 = jnp.dot(h, w2_ref[...], preferred_element_type=jnp.float32)
    z2_ref[...] = z2
    g2_ref[...] = z2 * dcol


_mid_call = pl.pallas_call(
    _mid_body,
    grid=(GRID,),
    in_specs=[
        pl.BlockSpec((R, H), lambda i: (i, 0)),
        pl.BlockSpec((NC, R, H), lambda i: (0, i, 0)),
        pl.BlockSpec((R,), lambda i: (i,)),
        pl.BlockSpec((1, H), lambda i: (0, 0)),
        pl.BlockSpec((H, H), lambda i: (0, 0)),
    ],
    out_specs=[
        pl.BlockSpec((R, H), lambda i: (i, 0)),
        pl.BlockSpec((R, H), lambda i: (i, 0)),
    ],
    out_shape=[
        jax.ShapeDtypeStruct((NN, H), jnp.float32),
        jax.ShapeDtypeStruct((NN, H), jnp.float32),
    ],
)


def _post_body(z_ref, p_ref, dinv_ref, b2_ref, p1_ref, pb1_ref, p2_ref, pb2_ref, out_ref):
    dcol = _col(dinv_ref[...])
    scat = p_ref[0] + p_ref[1]
    h = jnp.maximum(dcol * scat + dcol * dcol * z_ref[...] + b2_ref[...], 0.0)
    hidden = jnp.maximum(
        jnp.dot(h, p1_ref[...], preferred_element_type=jnp.float32) + pb1_ref[...], 0.0
    )
    out_ref[...] = (
        jnp.dot(hidden, p2_ref[...], preferred_element_type=jnp.float32) + pb2_ref[...]
    )


_post_call = pl.pallas_call(
    _post_body,
    grid=(GRID,),
    in_specs=[
        pl.BlockSpec((R, H), lambda i: (i, 0)),
        pl.BlockSpec((NC, R, H), lambda i: (0, i, 0)),
        pl.BlockSpec((R,), lambda i: (i,)),
        pl.BlockSpec((1, H), lambda i: (0, 0)),
        pl.BlockSpec((H, 512), lambda i: (0, 0)),
        pl.BlockSpec((1, 512), lambda i: (0, 0)),
        pl.BlockSpec((512, A), lambda i: (0, 0)),
        pl.BlockSpec((1, A), lambda i: (0, 0)),
    ],
    out_specs=pl.BlockSpec((R, A), lambda i: (i, 0)),
    out_shape=jax.ShapeDtypeStruct((N, A), jnp.float32),
)


def kernel(features, edge_index, W1, b1, W2, b2, P1, pb1, P2, pb2):
    src3 = edge_index[0].reshape(W, NCHUNK, CH)
    dst3 = edge_index[1].reshape(W, NCHUNK, CH)
    zeros1 = jnp.zeros((NN,), jnp.float32)
    zerosH = jnp.zeros((NN, H), jnp.float32)

    dpart = _sc_deg(dst3, zeros1)
    z1 = _z1_call(features, W1)
    dinv, g1 = _pre_call(z1, dpart)
    pp1 = _sc_prop(src3, dst3, g1, zerosH)
    z2, g2 = _mid_call(z1, pp1, dinv, b1.reshape(1, H), W2)
    pp2 = _sc_prop(src3, dst3, g2, zerosH)
    policy = _post_call(
        z2, pp2, dinv, b2.reshape(1, H),
        P1, pb1.reshape(1, 512), P2, pb2.reshape(1, A),
    )
    return policy


# speedup vs baseline: 46.6968x; 1.0336x over previous
import functools

import jax
import jax.numpy as jnp
from jax import lax
from jax.experimental import pallas as pl
from jax.experimental.pallas import tpu as pltpu
from jax.experimental.pallas import tpu_sc as plsc

N = 10000
F = 128
H = 32
A = 64
E = 320000

NC = 2
NS = 16
L = 16
W = NC * NS

CH = 125
NCHUNK = 80
EPT = NCHUNK * CH
NB = 4

NN = 10240
NPK = NN // 4
NPS = NN // NS

_mesh = plsc.VectorSubcoreMesh(
    core_axis_name="c", subcore_axis_name="s", num_cores=NC, num_subcores=NS
)


@functools.partial(
    pl.kernel,
    out_type=jax.ShapeDtypeStruct((NC, NN), jnp.float32),
    mesh=_mesh,
    scratch_types=[
        pltpu.VMEM((NCHUNK, CH), jnp.int32),
        pltpu.VMEM((128,), jnp.float32),
        pltpu.VMEM((NPS,), jnp.float32),
        pltpu.VMEM_SHARED((NN,), jnp.float32),
        pltpu.SemaphoreType.DMA,
    ],
    compiler_params=pltpu.CompilerParams(use_tc_tiling_on_sc=False),
)
def _sc_deg(dst_hbm, zeros_hbm, out_hbm, didx_v, ones_v, stage_v, accum_sh, sem):
    c = lax.axis_index("c")
    s = lax.axis_index("s")
    w = c * NS + s
    pltpu.sync_copy(dst_hbm.at[w], didx_v)
    for k in range(8):
        ones_v[pl.ds(k * L, L)] = jnp.full((L,), 1.0, jnp.float32)
    pltpu.sync_copy(zeros_hbm.at[pl.ds(s * NPS, NPS)], accum_sh.at[pl.ds(s * NPS, NPS)])
    plsc.subcore_barrier()

    def fire(j, carry):
        pltpu.async_copy(ones_v.at[pl.ds(0, CH)], accum_sh.at[didx_v.at[j]], sem, add=True)
        return carry

    lax.fori_loop(0, NCHUNK, fire, 0)

    def drain(j, carry):
        pltpu.make_async_copy(ones_v.at[pl.ds(0, CH)], accum_sh.at[didx_v.at[0]], sem).wait()
        return carry

    lax.fori_loop(0, NCHUNK, drain, 0)
    plsc.subcore_barrier()
    pltpu.sync_copy(accum_sh.at[pl.ds(s * NPS, NPS)], stage_v)
    pltpu.sync_copy(stage_v, out_hbm.at[c, pl.ds(s * NPS, NPS)])


@functools.partial(
    pl.kernel,
    out_type=jax.ShapeDtypeStruct((NC, NN, H), jnp.float32),
    mesh=_mesh,
    scratch_types=[
        pltpu.VMEM((NCHUNK, CH), jnp.int32),
        pltpu.VMEM((NCHUNK, CH), jnp.int32),
        pltpu.VMEM((CH, H), jnp.float32),
        pltpu.VMEM((CH, H), jnp.float32),
        pltpu.VMEM((CH, H), jnp.float32),
        pltpu.VMEM((CH, H), jnp.float32),
        pltpu.VMEM((NPS, H), jnp.float32),
        pltpu.VMEM_SHARED((NN, H), jnp.float32),
        pltpu.VMEM_SHARED((NN, H), jnp.float32),
        pltpu.SemaphoreType.DMA,
        pltpu.SemaphoreType.DMA,
        pltpu.SemaphoreType.DMA,
        pltpu.SemaphoreType.DMA,
        pltpu.SemaphoreType.DMA,
        pltpu.SemaphoreType.DMA,
        pltpu.SemaphoreType.DMA,
        pltpu.SemaphoreType.DMA,
    ],
    compiler_params=pltpu.CompilerParams(use_tc_tiling_on_sc=False),
)
def _sc_prop(src_hbm, dst_hbm, g_hbm, zeros_hbm, out_hbm,
             sidx_v, didx_v, r0, r1, r2, r3, stage_v, accum_sh, g_sh,
             g0, g1, g2, g3, s0, s1, s2, s3):
    c = lax.axis_index("c")
    s = lax.axis_index("s")
    w = c * NS + s
    bufs = (r0, r1, r2, r3)
    gsem = (g0, g1, g2, g3)
    ssem = (s0, s1, s2, s3)

    pltpu.sync_copy(src_hbm.at[w], sidx_v)
    pltpu.sync_copy(dst_hbm.at[w], didx_v)
    pltpu.sync_copy(zeros_hbm.at[pl.ds(s * NPS, NPS)], accum_sh.at[pl.ds(s * NPS, NPS)])
    pltpu.sync_copy(g_hbm.at[pl.ds(s * NPS, NPS)], g_sh.at[pl.ds(s * NPS, NPS)])
    plsc.subcore_barrier()

    for b in range(NB):
        pltpu.async_copy(g_sh.at[sidx_v.at[b]], bufs[b], gsem[b])

    def group(i, carry):
        j0 = NB * i
        for b in range(NB):
            pltpu.make_async_copy(g_sh.at[sidx_v.at[0]], bufs[b], gsem[b]).wait()
            pltpu.async_copy(bufs[b], accum_sh.at[didx_v.at[j0 + b]], ssem[b], add=True)
        for b in range(NB):
            pltpu.make_async_copy(bufs[b], accum_sh.at[didx_v.at[0]], ssem[b]).wait()
            pltpu.async_copy(g_sh.at[sidx_v.at[j0 + NB + b]], bufs[b], gsem[b])
        return carry

    lax.fori_loop(0, NCHUNK // NB - 1, group, 0)
    j0 = NCHUNK - NB
    for b in range(NB):
        pltpu.make_async_copy(g_sh.at[sidx_v.at[0]], bufs[b], gsem[b]).wait()
        pltpu.async_copy(bufs[b], accum_sh.at[didx_v.at[j0 + b]], ssem[b], add=True)
    for b in range(NB):
        pltpu.make_async_copy(bufs[b], accum_sh.at[didx_v.at[0]], ssem[b]).wait()

    plsc.subcore_barrier()
    pltpu.sync_copy(accum_sh.at[pl.ds(s * NPS, NPS)], stage_v)
    pltpu.sync_copy(stage_v, out_hbm.at[c, pl.ds(s * NPS, NPS)])


R = 256
GRID = NPK // R

_pk = pl.BlockSpec((R, 128), lambda p: (p, 0))
_pk3 = pl.BlockSpec((NC, R, 128), lambda p: (0, p, 0))


def _col(lane_vec):
    return jnp.transpose(lane_vec.reshape(1, R))


def _z1_body(x0_ref, x1_ref, x2_ref, x3_ref, w1p_ref, z_ref):
    xs = (x0_ref, x1_ref, x2_ref, x3_ref)
    acc = jnp.zeros((R, 128), jnp.float32)
    for k in range(4):
        acc = acc + jnp.dot(
            xs[k][...], w1p_ref[k], preferred_element_type=jnp.float32
        )
    z_ref[...] = acc


def _x_spec(k):
    return pl.BlockSpec((R, F), lambda p, k=k: (GRID * k + p, 0))


_z1_call = pl.pallas_call(
    _z1_body,
    grid=(GRID,),
    in_specs=[
        _x_spec(0), _x_spec(1), _x_spec(2), _x_spec(3),
        pl.BlockSpec((4, F, 128), lambda p: (0, 0, 0)),
    ],
    out_specs=_pk,
    out_shape=jax.ShapeDtypeStruct((NPK, 128), jnp.float32),
)


def _d_spec(k):
    return pl.BlockSpec((NC, R), lambda p, k=k: (0, GRID * k + p))


def _pre_body(z_ref, d0_ref, d1_ref, d2_ref, d3_ref, dm_ref, g_ref):
    ds = (d0_ref, d1_ref, d2_ref, d3_ref)
    lane = lax.broadcasted_iota(jnp.int32, (R, 128), 1) // H
    dm = jnp.zeros((R, 128), jnp.float32)
    for k in range(4):
        dinv_k = jax.lax.rsqrt(1.0 + ds[k][0] + ds[k][1])
        dm = dm + jnp.where(lane == k, _col(dinv_k), 0.0)
    dm_ref[...] = dm
    g_ref[...] = z_ref[...] * dm


_pre_call = pl.pallas_call(
    _pre_body,
    grid=(GRID,),
    in_specs=[_pk, _d_spec(0), _d_spec(1), _d_spec(2), _d_spec(3)],
    out_specs=[_pk, _pk],
    out_shape=[
        jax.ShapeDtypeStruct((NPK, 128), jnp.float32),
        jax.ShapeDtypeStruct((NPK, 128), jnp.float32),
    ],
)


def _mid_body(z_ref, p_ref, dm_ref, b1_ref, w2bd_ref, z2_ref, g2_ref):
    dm = dm_ref[...]
    scat = p_ref[0] + p_ref[1]
    h = jnp.maximum(dm * scat + dm * dm * z_ref[...] + b1_ref[...], 0.0)
    z2 = jnp.dot(h, w2bd_ref[...], preferred_element_type=jnp.float32)
    z2_ref[...] = z2
    g2_ref[...] = z2 * dm


_mid_call = pl.pallas_call(
    _mid_body,
    grid=(GRID,),
    in_specs=[
        _pk,
        _pk3,
        _pk,
        pl.BlockSpec((1, 128), lambda p: (0, 0)),
        pl.BlockSpec((128, 128), lambda p: (0, 0)),
    ],
    out_specs=[_pk, _pk],
    out_shape=[
        jax.ShapeDtypeStruct((NPK, 128), jnp.float32),
        jax.ShapeDtypeStruct((NPK, 128), jnp.float32),
    ],
)


def _post_body(z_ref, p_ref, dm_ref, b2_ref, p1e_ref, pb1_ref, p2_ref, pb2_ref, out_ref):
    dm = dm_ref[...]
    scat = p_ref[0] + p_ref[1]
    h = jnp.maximum(dm * scat + dm * dm * z_ref[...] + b2_ref[...], 0.0)
    for k in range(4):
        hidden = jnp.maximum(
            jnp.dot(h, p1e_ref[k], preferred_element_type=jnp.float32) + pb1_ref[...],
            0.0,
        )
        out_ref[k] = (
            jnp.dot(hidden, p2_ref[...], preferred_element_type=jnp.float32)
            + pb2_ref[...]
        )


_post_call = pl.pallas_call(
    _post_body,
    grid=(GRID,),
    in_specs=[
        _pk,
        _pk3,
        _pk,
        pl.BlockSpec((1, 128), lambda p: (0, 0)),
        pl.BlockSpec((4, 128, 512), lambda p: (0, 0, 0)),
        pl.BlockSpec((1, 512), lambda p: (0, 0)),
        pl.BlockSpec((512, A), lambda p: (0, 0)),
        pl.BlockSpec((1, A), lambda p: (0, 0)),
    ],
    out_specs=pl.BlockSpec((4, R, A), lambda p: (0, p, 0)),
    out_shape=jax.ShapeDtypeStruct((4, NPK, A), jnp.float32),
)


def kernel(features, edge_index, W1, b1, W2, b2, P1, pb1, P2, pb2):
    src = edge_index[0]
    dst = edge_index[1]
    src_p = 4 * (src % NPK) + src // NPK
    dst_p = 4 * (dst % NPK) + dst // NPK
    src3 = src_p.reshape(W, NCHUNK, CH)
    dst3p = dst_p.reshape(W, NCHUNK, CH)
    dst3 = dst.reshape(W, NCHUNK, CH)
    zeros1 = jnp.zeros((NN,), jnp.float32)
    zerosH = jnp.zeros((NN, H), jnp.float32)

    w1p = jnp.stack(
        [jnp.pad(W1, ((0, 0), (H * k, 128 - H * (k + 1)))) for k in range(4)]
    )
    w2bd = jax.scipy.linalg.block_diag(W2, W2, W2, W2)
    p1e = jnp.stack(
        [jnp.pad(P1, ((H * k, 128 - H * (k + 1)), (0, 0))) for k in range(4)]
    )
    b1t = jnp.tile(b1, 4).reshape(1, 128)
    b2t = jnp.tile(b2, 4).reshape(1, 128)

    dpart = _sc_deg(dst3, zeros1)
    z1 = _z1_call(features, features, features, features, w1p)
    dm, g1 = _pre_call(z1, dpart, dpart, dpart, dpart)
    pp1 = _sc_prop(src3, dst3p, g1.reshape(NN, H), zerosH).reshape(NC, NPK, 128)
    z2, g2 = _mid_call(z1, pp1, dm, b1t, w2bd)
    pp2 = _sc_prop(src3, dst3p, g2.reshape(NN, H), zerosH).reshape(NC, NPK, 128)
    pol4 = _post_call(z2, pp2, dm, b2t, p1e, pb1.reshape(1, 512), P2, pb2.reshape(1, A))
    return pol4.reshape(NN, A)[:N]
